# baseline (speedup 1.0000x reference)
import functools

import jax
import jax.numpy as jnp
from jax import lax
from jax.experimental import pallas as pl
from jax.experimental.pallas import tpu as pltpu
from jax.experimental.pallas import tpu_sc as plsc

N_NODES = 10000
N_EDGES = 320000
D_IN = 128
D_OUT = 128

N_TILES = 32
COLS_PER_TILE = D_OUT // N_TILES
CHUNK = 2000
N_CHUNKS = N_EDGES // CHUNK
GROUPS_PER_CHUNK = CHUNK // 16
UNROLL = 1


def _mm_body(w_ref, x_ref, out_ref):
    out_ref[...] = lax.dot_general(
        w_ref[...], x_ref[...], (((1,), (1,)), ((), ())),
        preferred_element_type=jnp.float32)


def _matmul_T(W, x):
    return pl.pallas_call(
        _mm_body,
        out_shape=jax.ShapeDtypeStruct((D_OUT, N_NODES), jnp.float32),
    )(W, x)


def _tr_body(a_ref, out_ref):
    out_ref[...] = a_ref[...].T


def _transpose_back(aT):
    return pl.pallas_call(
        _tr_body,
        out_shape=jax.ShapeDtypeStruct((N_NODES, D_OUT), jnp.float32),
    )(aT)


def _sc_body(yT_hbm, src_hbm, tgt_hbm, w_hbm, outT_hbm,
             ycols, acc, src0, tgt0, w0, src1, tgt1, w1, sem0, sem1):
    wid = lax.axis_index("s") * 2 + lax.axis_index("c")
    base_word = wid * COLS_PER_TILE * N_NODES
    bufs = ((src0, tgt0, w0, sem0), (src1, tgt1, w1, sem1))

    def _issue(g, b):
        sv, tv, wv, sem = bufs[b]
        base = pl.multiple_of(g * CHUNK, 8)
        pltpu.async_copy(src_hbm.at[pl.ds(base, CHUNK)], sv, sem)
        pltpu.async_copy(tgt_hbm.at[pl.ds(base, CHUNK)], tv, sem)
        pltpu.async_copy(w_hbm.at[pl.ds(base, CHUNK)], wv, sem)

    def _drain(b):
        sv, tv, wv, sem = bufs[b]
        pltpu.make_async_copy(src_hbm.at[pl.ds(0, CHUNK)], sv, sem).wait()
        pltpu.make_async_copy(tgt_hbm.at[pl.ds(0, CHUNK)], tv, sem).wait()
        pltpu.make_async_copy(w_hbm.at[pl.ds(0, CHUNK)], wv, sem).wait()

    def _process(b):
        sv, tv, wv, _ = bufs[b]

        @plsc.parallel_loop(0, GROUPS_PER_CHUNK, unroll=UNROLL)
        def _group(j):
            off = j * 16
            s16 = sv[pl.ds(off, 16)]
            t16 = tv[pl.ds(off, 16)]
            w16 = wv[pl.ds(off, 16)]
            for c in range(COLS_PER_TILE):
                coff = c * N_NODES
                vals = plsc.load_gather(ycols, [s16 + coff]) * w16
                plsc.addupdate_scatter(acc, [t16 + coff], vals)

    pltpu.async_copy(yT_hbm.at[pl.ds(base_word, COLS_PER_TILE * N_NODES)],
                     ycols, sem1)
    _issue(0, 0)

    @plsc.parallel_loop(0, COLS_PER_TILE * N_NODES // 16, unroll=8)
    def _zero(i):
        acc[pl.ds(i * 16, 16)] = jnp.zeros((16,), jnp.float32)

    pltpu.make_async_copy(
        yT_hbm.at[pl.ds(base_word, COLS_PER_TILE * N_NODES)], ycols,
        sem1).wait()

    def _two_chunks(h, _):
        g0 = h * 2
        _drain(0)
        _issue(g0 + 1, 1)
        _process(0)
        _drain(1)

        @pl.when(g0 + 2 < N_CHUNKS)
        def _():
            _issue(g0 + 2, 0)
        _process(1)
        return 0
    lax.fori_loop(0, N_CHUNKS // 2, _two_chunks, 0)

    pltpu.sync_copy(acc, outT_hbm.at[pl.ds(base_word,
                                           COLS_PER_TILE * N_NODES)])


def _sc_scatter(yT, source, target, edge_weights):
    mesh = plsc.VectorSubcoreMesh(core_axis_name="c", subcore_axis_name="s")
    f = functools.partial(
        pl.kernel,
        out_type=jax.ShapeDtypeStruct((D_OUT * N_NODES,), jnp.float32),
        mesh=mesh,
        compiler_params=pltpu.CompilerParams(needs_layout_passes=False),
        scratch_types=[
            pltpu.VMEM((COLS_PER_TILE * N_NODES,), jnp.float32),
            pltpu.VMEM((COLS_PER_TILE * N_NODES,), jnp.float32),
            pltpu.VMEM((CHUNK,), jnp.int32),
            pltpu.VMEM((CHUNK,), jnp.int32),
            pltpu.VMEM((CHUNK,), jnp.float32),
            pltpu.VMEM((CHUNK,), jnp.int32),
            pltpu.VMEM((CHUNK,), jnp.int32),
            pltpu.VMEM((CHUNK,), jnp.float32),
            pltpu.SemaphoreType.DMA,
            pltpu.SemaphoreType.DMA,
        ],
    )(_sc_body)
    accT_flat = f(yT.reshape(-1), source, target, edge_weights)
    return accT_flat.reshape(D_OUT, N_NODES)


def kernel(x, source, target, edge_weights, W):
    yT = _matmul_T(W, x)
    accT = _sc_scatter(yT, source, target, edge_weights)
    return _transpose_back(accT)


# speedup vs baseline: 1.0173x; 1.0173x over previous
import functools

import jax
import jax.numpy as jnp
from jax import lax
from jax.experimental import pallas as pl
from jax.experimental.pallas import tpu as pltpu
from jax.experimental.pallas import tpu_sc as plsc

N_NODES = 10000
N_EDGES = 320000
D_IN = 128
D_OUT = 128

N_TILES = 32
COLS_PER_TILE = D_OUT // N_TILES
CHUNK = 8000
N_CHUNKS = N_EDGES // CHUNK
GROUPS_PER_CHUNK = CHUNK // 16
UNROLL = 1


def _mm_body(w_ref, x_ref, out_ref):
    out_ref[...] = lax.dot_general(
        w_ref[...], x_ref[...], (((1,), (1,)), ((), ())),
        preferred_element_type=jnp.float32)


def _matmul_T(W, x):
    return pl.pallas_call(
        _mm_body,
        out_shape=jax.ShapeDtypeStruct((D_OUT, N_NODES), jnp.float32),
    )(W, x)


def _tr_body(a_ref, out_ref):
    out_ref[...] = a_ref[...].T


def _transpose_back(aT):
    return pl.pallas_call(
        _tr_body,
        out_shape=jax.ShapeDtypeStruct((N_NODES, D_OUT), jnp.float32),
    )(aT)


def _sc_body(yT_hbm, src_hbm, tgt_hbm, w_hbm, outT_hbm,
             ycols, acc, src0, tgt0, w0, src1, tgt1, w1, sem0, sem1):
    wid = lax.axis_index("s") * 2 + lax.axis_index("c")
    base_word = wid * COLS_PER_TILE * N_NODES
    bufs = ((src0, tgt0, w0, sem0), (src1, tgt1, w1, sem1))

    def _issue(g, b):
        sv, tv, wv, sem = bufs[b]
        base = pl.multiple_of(g * CHUNK, 8)
        pltpu.async_copy(src_hbm.at[pl.ds(base, CHUNK)], sv, sem)
        pltpu.async_copy(tgt_hbm.at[pl.ds(base, CHUNK)], tv, sem)
        pltpu.async_copy(w_hbm.at[pl.ds(base, CHUNK)], wv, sem)

    def _drain(b):
        sv, tv, wv, sem = bufs[b]
        pltpu.make_async_copy(src_hbm.at[pl.ds(0, CHUNK)], sv, sem).wait()
        pltpu.make_async_copy(tgt_hbm.at[pl.ds(0, CHUNK)], tv, sem).wait()
        pltpu.make_async_copy(w_hbm.at[pl.ds(0, CHUNK)], wv, sem).wait()

    def _process(b):
        sv, tv, wv, _ = bufs[b]

        @plsc.parallel_loop(0, GROUPS_PER_CHUNK, unroll=UNROLL)
        def _group(j):
            off = j * 16
            s16 = sv[pl.ds(off, 16)]
            t16 = tv[pl.ds(off, 16)]
            w16 = wv[pl.ds(off, 16)]
            for c in range(COLS_PER_TILE):
                coff = c * N_NODES
                vals = plsc.load_gather(ycols, [s16 + coff]) * w16
                plsc.addupdate_scatter(acc, [t16 + coff], vals)

    pltpu.async_copy(yT_hbm.at[pl.ds(base_word, COLS_PER_TILE * N_NODES)],
                     ycols, sem1)
    _issue(0, 0)

    @plsc.parallel_loop(0, COLS_PER_TILE * N_NODES // 16, unroll=8)
    def _zero(i):
        acc[pl.ds(i * 16, 16)] = jnp.zeros((16,), jnp.float32)

    pltpu.make_async_copy(
        yT_hbm.at[pl.ds(base_word, COLS_PER_TILE * N_NODES)], ycols,
        sem1).wait()

    def _two_chunks(h, _):
        g0 = h * 2
        _drain(0)
        _issue(g0 + 1, 1)
        _process(0)
        _drain(1)

        @pl.when(g0 + 2 < N_CHUNKS)
        def _():
            _issue(g0 + 2, 0)
        _process(1)
        return 0
    lax.fori_loop(0, N_CHUNKS // 2, _two_chunks, 0)

    pltpu.sync_copy(acc, outT_hbm.at[pl.ds(base_word,
                                           COLS_PER_TILE * N_NODES)])


def _sc_scatter(yT, source, target, edge_weights):
    mesh = plsc.VectorSubcoreMesh(core_axis_name="c", subcore_axis_name="s")
    f = functools.partial(
        pl.kernel,
        out_type=jax.ShapeDtypeStruct((D_OUT * N_NODES,), jnp.float32),
        mesh=mesh,
        compiler_params=pltpu.CompilerParams(needs_layout_passes=False),
        scratch_types=[
            pltpu.VMEM((COLS_PER_TILE * N_NODES,), jnp.float32),
            pltpu.VMEM((COLS_PER_TILE * N_NODES,), jnp.float32),
            pltpu.VMEM((CHUNK,), jnp.int32),
            pltpu.VMEM((CHUNK,), jnp.int32),
            pltpu.VMEM((CHUNK,), jnp.float32),
            pltpu.VMEM((CHUNK,), jnp.int32),
            pltpu.VMEM((CHUNK,), jnp.int32),
            pltpu.VMEM((CHUNK,), jnp.float32),
            pltpu.SemaphoreType.DMA,
            pltpu.SemaphoreType.DMA,
        ],
    )(_sc_body)
    accT_flat = f(yT.reshape(-1), source, target, edge_weights)
    return accT_flat.reshape(D_OUT, N_NODES)


def kernel(x, source, target, edge_weights, W):
    yT = _matmul_T(W, x)
    accT = _sc_scatter(yT, source, target, edge_weights)
    return _transpose_back(accT)
